# Initial kernel scaffold; baseline (speedup 1.0000x reference)
import jax
import jax.numpy as jnp
from jax.experimental import pallas as pl


def kernel(x, W, b, leaves):
    raise NotImplementedError("write your pallas kernel here")


# speedup vs baseline: 57.5175x; 57.5175x over previous
import functools

import jax
import jax.numpy as jnp
import numpy as np
from jax import lax
from jax.experimental import pallas as pl
from jax.experimental.pallas import tpu as pltpu
from jax.experimental.pallas import tpu_sc as plsc

INPUT_DIM = 128
NUM_CLASSES = 64
NUM_TREES = 100
TREE_DEPTH = 5
NUM_USED_NODES = 16
NUM_LEAVES = 32
BATCH = 4096

SAMPLE_BLOCK = 512
TC_GRID = BATCH // SAMPLE_BLOCK

NUM_CORES = 2
NUM_SUBCORES = 16
NUM_WORKERS = NUM_CORES * NUM_SUBCORES
ROWS_PER_WORKER = BATCH // NUM_WORKERS
LANES = 16
GROUPS = ROWS_PER_WORKER // LANES
CLASS_CHUNKS = NUM_CLASSES // LANES

_FLAT_NODES = NUM_TREES * NUM_USED_NODES


def _mask_kernel(x_ref, w_ref, b_ref, p_ref, out_ref):
    logits = lax.dot_general(
        x_ref[...], w_ref[...], (((1,), (1,)), ((), ())),
        precision=lax.Precision.HIGHEST)
    bits = jnp.where(logits + b_ref[...] <= 0.0, 1.0, 0.0)
    mask_f = lax.dot_general(
        bits, p_ref[...], (((1,), (0,)), ((), ())),
        precision=lax.Precision.HIGHEST)
    out_ref[...] = mask_f.astype(jnp.int32)


def _compute_masks(x, w_flat, b_flat, pack):
    return pl.pallas_call(
        _mask_kernel,
        grid=(TC_GRID,),
        in_specs=[
            pl.BlockSpec((SAMPLE_BLOCK, INPUT_DIM), lambda i: (i, 0)),
            pl.BlockSpec((_FLAT_NODES, INPUT_DIM), lambda i: (0, 0)),
            pl.BlockSpec((1, _FLAT_NODES), lambda i: (0, 0)),
            pl.BlockSpec((_FLAT_NODES, NUM_TREES), lambda i: (0, 0)),
        ],
        out_specs=pl.BlockSpec((SAMPLE_BLOCK, NUM_TREES), lambda i: (i, 0)),
        out_shape=jax.ShapeDtypeStruct((BATCH, NUM_TREES), jnp.int32),
    )(x, w_flat, b_flat, pack)


def _sc_body(masks_hbm, leaves_hbm, out_hbm, masks_v, idx_v, rows_v, acc_v,
             sem_in, sem_g):
    wid = lax.axis_index("s") * NUM_CORES + lax.axis_index("c")
    base = wid * ROWS_PER_WORKER

    pltpu.sync_copy(
        masks_hbm.at[pl.ds(base * NUM_TREES, ROWS_PER_WORKER * NUM_TREES)],
        masks_v)

    zero = jnp.zeros((LANES,), jnp.float32)
    for r in range(ROWS_PER_WORKER):
        for c in range(CLASS_CHUNKS):
            acc_v[r, pl.ds(c * LANES, LANES)] = zero

    lane_iota = lax.iota(jnp.int32, LANES)

    def tree_body(t, carry):
        for g in range(GROUPS):
            rows = g * LANES + lane_iota
            m = plsc.load_gather(masks_v, [rows * NUM_TREES + t])
            idx = jnp.zeros((LANES,), jnp.int32)
            for _ in range(TREE_DEPTH):
                bit = lax.shift_right_logical(m, idx) & 1
                idx = idx + idx + bit
            idx_v[pl.ds(g * LANES, LANES)] = t * NUM_LEAVES + idx
        pltpu.async_copy(leaves_hbm.at[idx_v], rows_v, sem_g).wait()
        for r in range(ROWS_PER_WORKER):
            for c in range(CLASS_CHUNKS):
                sl = pl.ds(c * LANES, LANES)
                plsc.addupdate(acc_v.at[r, sl], rows_v[r, sl])
        return carry

    lax.fori_loop(0, NUM_TREES, tree_body, 0)

    scale = jnp.full((LANES,), 1.0 / NUM_TREES, jnp.float32)
    for r in range(ROWS_PER_WORKER):
        for c in range(CLASS_CHUNKS):
            sl = pl.ds(c * LANES, LANES)
            acc_v[r, sl] = acc_v[r, sl] * scale
    pltpu.sync_copy(acc_v, out_hbm.at[pl.ds(base, ROWS_PER_WORKER), :])


@functools.cache
def _sc_forest():
    return pl.kernel(
        _sc_body,
        out_type=jax.ShapeDtypeStruct((BATCH, NUM_CLASSES), jnp.float32),
        mesh=plsc.VectorSubcoreMesh(
            core_axis_name="c", subcore_axis_name="s",
            num_cores=NUM_CORES, num_subcores=NUM_SUBCORES),
        compiler_params=pltpu.CompilerParams(
            needs_layout_passes=False, use_tc_tiling_on_sc=False),
        scratch_types=[
            pltpu.VMEM((ROWS_PER_WORKER * NUM_TREES,), jnp.int32),
            pltpu.VMEM((ROWS_PER_WORKER,), jnp.int32),
            pltpu.VMEM((ROWS_PER_WORKER, NUM_CLASSES), jnp.float32),
            pltpu.VMEM((ROWS_PER_WORKER, NUM_CLASSES), jnp.float32),
            pltpu.SemaphoreType.DMA,
            pltpu.SemaphoreType.DMA,
        ],
    )


def _pack_matrix():
    p = np.zeros((_FLAT_NODES, NUM_TREES), np.float32)
    for t in range(NUM_TREES):
        for n in range(NUM_USED_NODES):
            p[t * NUM_USED_NODES + n, t] = float(1 << n)
    return jnp.asarray(p)


def kernel(x, W, b, leaves):
    w_flat = W[:, :NUM_USED_NODES, :].reshape(_FLAT_NODES, INPUT_DIM)
    b_flat = b[:, :NUM_USED_NODES].reshape(1, _FLAT_NODES)
    masks = _compute_masks(x, w_flat, b_flat, _pack_matrix())
    leaves_flat = leaves.reshape(NUM_TREES * NUM_LEAVES, NUM_CLASSES)
    return _sc_forest()(masks.reshape(BATCH * NUM_TREES), leaves_flat)


# speedup vs baseline: 72.3039x; 1.2571x over previous
import functools

import jax
import jax.numpy as jnp
import numpy as np
from jax import lax
from jax.experimental import pallas as pl
from jax.experimental.pallas import tpu as pltpu
from jax.experimental.pallas import tpu_sc as plsc

INPUT_DIM = 128
NUM_CLASSES = 64
NUM_TREES = 100
TREE_DEPTH = 5
NUM_USED_NODES = 16
NUM_LEAVES = 32
BATCH = 4096

SAMPLE_BLOCK = 512
TC_GRID = BATCH // SAMPLE_BLOCK

NUM_CORES = 2
NUM_SUBCORES = 16
NUM_WORKERS = NUM_CORES * NUM_SUBCORES
ROWS_PER_WORKER = BATCH // NUM_WORKERS
LANES = 16
GROUPS = ROWS_PER_WORKER // LANES
CLASS_CHUNKS = NUM_CLASSES // LANES

_FLAT_NODES = NUM_TREES * NUM_USED_NODES


def _mask_kernel(x_ref, w_ref, b_ref, p_ref, out_ref):
    logits = lax.dot_general(
        x_ref[...], w_ref[...], (((1,), (1,)), ((), ())),
        precision=lax.Precision.HIGHEST)
    bits = jnp.where(logits + b_ref[...] <= 0.0, 1.0, 0.0)
    mask_f = lax.dot_general(
        bits, p_ref[...], (((1,), (0,)), ((), ())),
        precision=lax.Precision.HIGHEST)
    out_ref[...] = mask_f.astype(jnp.int32)


def _compute_masks(x, w_flat, b_flat, pack):
    return pl.pallas_call(
        _mask_kernel,
        grid=(TC_GRID,),
        in_specs=[
            pl.BlockSpec((SAMPLE_BLOCK, INPUT_DIM), lambda i: (i, 0)),
            pl.BlockSpec((_FLAT_NODES, INPUT_DIM), lambda i: (0, 0)),
            pl.BlockSpec((1, _FLAT_NODES), lambda i: (0, 0)),
            pl.BlockSpec((_FLAT_NODES, NUM_TREES), lambda i: (0, 0)),
        ],
        out_specs=pl.BlockSpec((SAMPLE_BLOCK, NUM_TREES), lambda i: (i, 0)),
        out_shape=jax.ShapeDtypeStruct((BATCH, NUM_TREES), jnp.int32),
    )(x, w_flat, b_flat, pack)


def _sc_body(masks_hbm, leaves_hbm, out_hbm, masks_v, idx_v0, idx_v1,
             rows_v0, rows_v1, acc_v, sem0, sem1):
    wid = lax.axis_index("s") * NUM_CORES + lax.axis_index("c")
    base = wid * ROWS_PER_WORKER

    pltpu.sync_copy(
        masks_hbm.at[pl.ds(base * NUM_TREES, ROWS_PER_WORKER * NUM_TREES)],
        masks_v)

    zero = jnp.zeros((LANES,), jnp.float32)
    for r in range(ROWS_PER_WORKER):
        for c in range(CLASS_CHUNKS):
            acc_v[r, pl.ds(c * LANES, LANES)] = zero

    lane_iota = lax.iota(jnp.int32, LANES)

    def route(t, idx_ref):
        for g in range(GROUPS):
            rows = g * LANES + lane_iota
            m = plsc.load_gather(masks_v, [rows * NUM_TREES + t])
            idx = jnp.zeros((LANES,), jnp.int32)
            for _ in range(TREE_DEPTH):
                bit = lax.shift_right_logical(m, idx) & 1
                idx = idx + idx + bit
            idx_ref[pl.ds(g * LANES, LANES)] = t * NUM_LEAVES + idx

    def accum(rows_ref):
        for r in range(ROWS_PER_WORKER):
            for c in range(CLASS_CHUNKS):
                sl = pl.ds(c * LANES, LANES)
                plsc.addupdate(acc_v.at[r, sl], rows_ref[r, sl])

    def issue(idx_ref, rows_ref, sem):
        pltpu.async_copy(leaves_hbm.at[idx_ref], rows_ref, sem)

    def drain(idx_ref, rows_ref, sem):
        pltpu.make_async_copy(leaves_hbm.at[idx_ref], rows_ref, sem).wait()

    route(0, idx_v0)
    issue(idx_v0, rows_v0, sem0)
    route(1, idx_v1)
    issue(idx_v1, rows_v1, sem1)

    def tree_pair(k, carry):
        t0 = 2 * k
        drain(idx_v0, rows_v0, sem0)
        accum(rows_v0)
        route(t0, idx_v0)
        issue(idx_v0, rows_v0, sem0)
        drain(idx_v1, rows_v1, sem1)
        accum(rows_v1)
        route(t0 + 1, idx_v1)
        issue(idx_v1, rows_v1, sem1)
        return carry

    lax.fori_loop(1, NUM_TREES // 2, tree_pair, 0)

    drain(idx_v0, rows_v0, sem0)
    accum(rows_v0)
    drain(idx_v1, rows_v1, sem1)
    accum(rows_v1)

    scale = jnp.full((LANES,), 1.0 / NUM_TREES, jnp.float32)
    for r in range(ROWS_PER_WORKER):
        for c in range(CLASS_CHUNKS):
            sl = pl.ds(c * LANES, LANES)
            acc_v[r, sl] = acc_v[r, sl] * scale
    pltpu.sync_copy(acc_v, out_hbm.at[pl.ds(base, ROWS_PER_WORKER), :])


@functools.cache
def _sc_forest():
    return pl.kernel(
        _sc_body,
        out_type=jax.ShapeDtypeStruct((BATCH, NUM_CLASSES), jnp.float32),
        mesh=plsc.VectorSubcoreMesh(
            core_axis_name="c", subcore_axis_name="s",
            num_cores=NUM_CORES, num_subcores=NUM_SUBCORES),
        compiler_params=pltpu.CompilerParams(
            needs_layout_passes=False, use_tc_tiling_on_sc=False),
        scratch_types=[
            pltpu.VMEM((ROWS_PER_WORKER * NUM_TREES,), jnp.int32),
            pltpu.VMEM((ROWS_PER_WORKER,), jnp.int32),
            pltpu.VMEM((ROWS_PER_WORKER,), jnp.int32),
            pltpu.VMEM((ROWS_PER_WORKER, NUM_CLASSES), jnp.float32),
            pltpu.VMEM((ROWS_PER_WORKER, NUM_CLASSES), jnp.float32),
            pltpu.VMEM((ROWS_PER_WORKER, NUM_CLASSES), jnp.float32),
            pltpu.SemaphoreType.DMA,
            pltpu.SemaphoreType.DMA,
        ],
    )


def _pack_matrix():
    p = np.zeros((_FLAT_NODES, NUM_TREES), np.float32)
    for t in range(NUM_TREES):
        for n in range(NUM_USED_NODES):
            p[t * NUM_USED_NODES + n, t] = float(1 << n)
    return jnp.asarray(p)


def kernel(x, W, b, leaves):
    w_flat = W[:, :NUM_USED_NODES, :].reshape(_FLAT_NODES, INPUT_DIM)
    b_flat = b[:, :NUM_USED_NODES].reshape(1, _FLAT_NODES)
    masks = _compute_masks(x, w_flat, b_flat, _pack_matrix())
    leaves_flat = leaves.reshape(NUM_TREES * NUM_LEAVES, NUM_CLASSES)
    return _sc_forest()(masks.reshape(BATCH * NUM_TREES), leaves_flat)


# speedup vs baseline: 126.3463x; 1.7474x over previous
import functools

import jax
import jax.numpy as jnp
import numpy as np
from jax import lax
from jax.experimental import pallas as pl
from jax.experimental.pallas import tpu as pltpu
from jax.experimental.pallas import tpu_sc as plsc

INPUT_DIM = 128
NUM_CLASSES = 64
NUM_TREES = 100
TREE_DEPTH = 5
NUM_USED_NODES = 16
NUM_LEAVES = 32
BATCH = 4096

SAMPLE_BLOCK = 512
TC_GRID = BATCH // SAMPLE_BLOCK

NUM_CORES = 2
NUM_SUBCORES = 16
NUM_WORKERS = NUM_CORES * NUM_SUBCORES
ROWS_PER_WORKER = BATCH // NUM_WORKERS
LANES = 16
GROUPS = ROWS_PER_WORKER // LANES
CLASS_CHUNKS = NUM_CLASSES // LANES

_FLAT_NODES = NUM_TREES * NUM_USED_NODES


def _mask_kernel(x_ref, w_ref, b_ref, p_ref, out_ref):
    logits = lax.dot_general(
        x_ref[...], w_ref[...], (((1,), (1,)), ((), ())),
        precision=lax.Precision.HIGHEST)
    bits = jnp.where(logits + b_ref[...] <= 0.0, 1.0, 0.0)
    mask_f = lax.dot_general(
        bits, p_ref[...], (((1,), (0,)), ((), ())),
        precision=lax.Precision.HIGHEST)
    out_ref[...] = mask_f.astype(jnp.int32)


def _compute_masks(x, w_flat, b_flat, pack):
    return pl.pallas_call(
        _mask_kernel,
        grid=(TC_GRID,),
        in_specs=[
            pl.BlockSpec((SAMPLE_BLOCK, INPUT_DIM), lambda i: (i, 0)),
            pl.BlockSpec((_FLAT_NODES, INPUT_DIM), lambda i: (0, 0)),
            pl.BlockSpec((1, _FLAT_NODES), lambda i: (0, 0)),
            pl.BlockSpec((_FLAT_NODES, NUM_TREES), lambda i: (0, 0)),
        ],
        out_specs=pl.BlockSpec((SAMPLE_BLOCK, NUM_TREES), lambda i: (i, 0)),
        out_shape=jax.ShapeDtypeStruct((BATCH, NUM_TREES), jnp.int32),
    )(x, w_flat, b_flat, pack)


def _sc_body(masks_hbm, leaves_hbm, out_hbm, masks_v, idx_all, acc_v, sem0):
    wid = lax.axis_index("s") * NUM_CORES + lax.axis_index("c")
    base = wid * ROWS_PER_WORKER

    pltpu.sync_copy(
        masks_hbm.at[pl.ds(base * NUM_TREES, ROWS_PER_WORKER * NUM_TREES)],
        masks_v)

    zero = jnp.zeros((LANES,), jnp.float32)
    for r in range(ROWS_PER_WORKER):
        for c in range(CLASS_CHUNKS):
            acc_v[r, pl.ds(c * LANES, LANES)] = zero

    lane_iota = lax.iota(jnp.int32, LANES)

    def tree_route(t, carry):
        for g in range(GROUPS):
            rows = g * LANES + lane_iota
            m = plsc.load_gather(masks_v, [rows * NUM_TREES + t])
            idx = jnp.zeros((LANES,), jnp.int32)
            for _ in range(TREE_DEPTH):
                bit = lax.shift_right_logical(m, idx) & 1
                idx = idx + idx + bit
            idx_all[pl.ds(t * ROWS_PER_WORKER + g * LANES, LANES)] = (
                t * NUM_LEAVES + idx)
        pltpu.async_copy(
            leaves_hbm.at[idx_all.at[pl.ds(t * ROWS_PER_WORKER,
                                           ROWS_PER_WORKER)]],
            acc_v, sem0, add=True)
        return carry

    lax.fori_loop(0, NUM_TREES, tree_route, 0)

    def drain(t, carry):
        pltpu.make_async_copy(
            leaves_hbm.at[idx_all.at[pl.ds(0, ROWS_PER_WORKER)]],
            acc_v, sem0).wait()
        return carry

    lax.fori_loop(0, NUM_TREES, drain, 0)

    scale = jnp.full((LANES,), 1.0 / NUM_TREES, jnp.float32)
    for r in range(ROWS_PER_WORKER):
        for c in range(CLASS_CHUNKS):
            sl = pl.ds(c * LANES, LANES)
            acc_v[r, sl] = acc_v[r, sl] * scale
    pltpu.sync_copy(acc_v, out_hbm.at[pl.ds(base, ROWS_PER_WORKER), :])


@functools.cache
def _sc_forest():
    return pl.kernel(
        _sc_body,
        out_type=jax.ShapeDtypeStruct((BATCH, NUM_CLASSES), jnp.float32),
        mesh=plsc.VectorSubcoreMesh(
            core_axis_name="c", subcore_axis_name="s",
            num_cores=NUM_CORES, num_subcores=NUM_SUBCORES),
        compiler_params=pltpu.CompilerParams(
            needs_layout_passes=False, use_tc_tiling_on_sc=False),
        scratch_types=[
            pltpu.VMEM((ROWS_PER_WORKER * NUM_TREES,), jnp.int32),
            pltpu.VMEM((NUM_TREES * ROWS_PER_WORKER,), jnp.int32),
            pltpu.VMEM((ROWS_PER_WORKER, NUM_CLASSES), jnp.float32),
            pltpu.SemaphoreType.DMA,
        ],
    )


def _pack_matrix():
    p = np.zeros((_FLAT_NODES, NUM_TREES), np.float32)
    for t in range(NUM_TREES):
        for n in range(NUM_USED_NODES):
            p[t * NUM_USED_NODES + n, t] = float(1 << n)
    return jnp.asarray(p)


def kernel(x, W, b, leaves):
    w_flat = W[:, :NUM_USED_NODES, :].reshape(_FLAT_NODES, INPUT_DIM)
    b_flat = b[:, :NUM_USED_NODES].reshape(1, _FLAT_NODES)
    masks = _compute_masks(x, w_flat, b_flat, _pack_matrix())
    leaves_flat = leaves.reshape(NUM_TREES * NUM_LEAVES, NUM_CLASSES)
    return _sc_forest()(masks.reshape(BATCH * NUM_TREES), leaves_flat)
